# baseline (speedup 1.0000x reference)
import functools

import jax
import jax.numpy as jnp
from jax import lax
from jax.experimental import pallas as pl
from jax.experimental.pallas import tpu as pltpu

B, M, N = 64, 32, 8192
K = 16


def _select_kernel(g_ref, s_ref, eg_ref, vals_ref, idx_ref, sel_ref, s0p_ref):
    x = g_ref[...]
    iota = lax.broadcasted_iota(jnp.int32, (B, N), 1)
    vals = []
    idxs = []
    for _ in range(K):
        m = jnp.max(x, axis=1, keepdims=True)
        idx = jnp.min(jnp.where(x >= m, iota, N), axis=1, keepdims=True)
        vals.append(m)
        idxs.append(idx)
        x = jnp.where(iota == idx, -1.0, x)
    vals_ref[...] = jnp.maximum(jnp.concatenate(vals, axis=1), 1e-8)
    idx_ref[...] = jnp.concatenate(idxs, axis=1)

    sp = s_ref[...]
    sc = sp * eg_ref[...]
    sm = jnp.max(sc, axis=1, keepdims=True)
    iota2 = lax.broadcasted_iota(jnp.int32, (K, N), 1)
    sel = jnp.min(jnp.where(sc >= sm, iota2, N), axis=1)
    sel_ref[0, :] = sel
    s0p_ref[0, :] = jnp.sum(jnp.where(iota2 == sel[:, None], sp, 0.0), axis=1)


def _gather_kernel(sel_ref, p_ref, out_ref):
    i = pl.program_id(0)
    s = sel_ref[i]
    row = (s // 64) % 8
    col = s % 64
    blk = p_ref[...]
    rmask = lax.broadcasted_iota(jnp.int32, (B, 8, 64), 1) == row
    cmask = lax.broadcasted_iota(jnp.int32, (B, 8, 64), 2) == col
    v = jnp.sum(jnp.where(rmask & cmask, blk, 0.0), axis=(1, 2))
    out_ref[0, 0, :] = jnp.maximum(v, 1e-8)


@jax.jit
def kernel(probs):
    eg = jnp.exp(jax.random.gumbel(jax.random.key(42), (K, N), jnp.float32))
    g2 = probs[:, 0, :]
    s2 = probs[0, K:, :]

    vals, idx, sel2d, s0p = pl.pallas_call(
        _select_kernel,
        grid=(),
        in_specs=[
            pl.BlockSpec((B, N), lambda: (0, 0)),
            pl.BlockSpec((K, N), lambda: (0, 0)),
            pl.BlockSpec((K, N), lambda: (0, 0)),
        ],
        out_specs=[
            pl.BlockSpec((B, K), lambda: (0, 0)),
            pl.BlockSpec((B, K), lambda: (0, 0)),
            pl.BlockSpec((1, K), lambda: (0, 0)),
            pl.BlockSpec((1, K), lambda: (0, 0)),
        ],
        out_shape=[
            jax.ShapeDtypeStruct((B, K), jnp.float32),
            jax.ShapeDtypeStruct((B, K), jnp.int32),
            jax.ShapeDtypeStruct((1, K), jnp.int32),
            jax.ShapeDtypeStruct((1, K), jnp.float32),
        ],
    )(g2, s2, eg)

    sel = sel2d[0]

    grid_spec = pltpu.PrefetchScalarGridSpec(
        num_scalar_prefetch=1,
        grid=(K,),
        in_specs=[
            pl.BlockSpec(
                (B, 8, 64),
                lambda i, sr: (0, 256 + 16 * i + sr[i] // 512, 0)),
        ],
        out_specs=pl.BlockSpec((1, 1, B), lambda i, sr: (i, 0, 0)),
    )
    sprobs = pl.pallas_call(
        _gather_kernel,
        grid_spec=grid_spec,
        out_shape=jax.ShapeDtypeStruct((K, 1, B), jnp.float32),
    )(sel, probs.reshape(B, M * N // 64, 64))

    selected = jnp.concatenate(
        [idx, jnp.broadcast_to(sel[None, :], (B, K))], axis=1)
    prob = jnp.concatenate([vals, sprobs[:, 0, :].T], axis=1)
    return selected, prob


# speedup vs baseline: 3.5335x; 3.5335x over previous
import functools

import jax
import jax.numpy as jnp
from jax import lax
from jax.experimental import pallas as pl
from jax.experimental.pallas import tpu as pltpu

B, M, N = 64, 32, 8192
K = 16


def _select_kernel(g_ref, s_ref, eg_ref, vals_ref, idx_ref, sel_ref, s0p_ref):
    x = g_ref[...]
    iota = lax.broadcasted_iota(jnp.int32, (B, N), 1)
    vals = []
    idxs = []
    for _ in range(K):
        m = jnp.max(x, axis=1, keepdims=True)
        idx = jnp.min(jnp.where(x >= m, iota, N), axis=1, keepdims=True)
        vals.append(m)
        idxs.append(idx)
        x = jnp.where(iota == idx, -1.0, x)
    vals_ref[...] = jnp.maximum(jnp.concatenate(vals, axis=1), 1e-8)
    idx_ref[...] = jnp.concatenate(idxs, axis=1)

    sp = s_ref[...]
    sc = sp * eg_ref[...]
    sm = jnp.max(sc, axis=1, keepdims=True)
    iota2 = lax.broadcasted_iota(jnp.int32, (K, N), 1)
    sel = jnp.min(jnp.where(sc >= sm, iota2, N), axis=1)
    sel_ref[0, :] = sel
    s0p_ref[0, :] = jnp.sum(jnp.where(iota2 == sel[:, None], sp, 0.0), axis=1)


def _gather_kernel(sel_ref, p_ref, out_ref):
    i = pl.program_id(0)
    row = i % 8
    col = sel_ref[i] % 128
    blk = p_ref[...]
    rmask = lax.broadcasted_iota(jnp.int32, (B, 8, 128), 1) == row
    cmask = lax.broadcasted_iota(jnp.int32, (B, 8, 128), 2) == col
    v = jnp.sum(jnp.where(rmask & cmask, blk, 0.0), axis=(1, 2))
    out_ref[0, 0, :] = jnp.maximum(v, 1e-8)


@jax.jit
def kernel(probs):
    eg = jnp.exp(jax.random.gumbel(jax.random.key(42), (K, N), jnp.float32))
    g2 = probs[:, 0, :]
    s2 = probs[0, K:, :]

    vals, idx, sel2d, s0p = pl.pallas_call(
        _select_kernel,
        grid=(),
        in_specs=[
            pl.BlockSpec((B, N), lambda: (0, 0)),
            pl.BlockSpec((K, N), lambda: (0, 0)),
            pl.BlockSpec((K, N), lambda: (0, 0)),
        ],
        out_specs=[
            pl.BlockSpec((B, K), lambda: (0, 0)),
            pl.BlockSpec((B, K), lambda: (0, 0)),
            pl.BlockSpec((1, K), lambda: (0, 0)),
            pl.BlockSpec((1, K), lambda: (0, 0)),
        ],
        out_shape=[
            jax.ShapeDtypeStruct((B, K), jnp.float32),
            jax.ShapeDtypeStruct((B, K), jnp.int32),
            jax.ShapeDtypeStruct((1, K), jnp.int32),
            jax.ShapeDtypeStruct((1, K), jnp.float32),
        ],
    )(g2, s2, eg)

    sel = sel2d[0]

    grid_spec = pltpu.PrefetchScalarGridSpec(
        num_scalar_prefetch=1,
        grid=(K,),
        in_specs=[
            pl.BlockSpec(
                (B, 8, 128),
                lambda i, sr: (0, 2 + i // 8, sr[i] // 128)),
        ],
        out_specs=pl.BlockSpec((1, 1, B), lambda i, sr: (i, 0, 0)),
    )
    sprobs = pl.pallas_call(
        _gather_kernel,
        grid_spec=grid_spec,
        out_shape=jax.ShapeDtypeStruct((K, 1, B), jnp.float32),
    )(sel, probs)

    selected = jnp.concatenate(
        [idx, jnp.broadcast_to(sel[None, :], (B, K))], axis=1)
    prob = jnp.concatenate([vals, sprobs[:, 0, :].T], axis=1)
    return selected, prob
